# baseline (device time: 49937 ns/iter reference)
import jax
import jax.numpy as jnp
from jax import lax
from jax.experimental import pallas as pl
from jax.experimental.pallas import tpu as pltpu

N_DEV = 16
M = 768
N_OUT = 768
CHUNK = M // N_DEV
N_BLOCKS = 4
BLOCK = M // N_BLOCKS
CPB = N_DEV // N_BLOCKS


def kernel(x, Wg, Wu, Wd):
    def body(x_ref, wg_ref, wu_ref, wd_ref, out_ref,
             stage, rs_buf, ag_buf,
             rs_send, rs_recv, ag_send, ag_recv):
        me = lax.axis_index("i")

        barrier = pltpu.get_barrier_semaphore()
        for j in range(1, N_DEV):
            pl.semaphore_signal(
                barrier, inc=1, device_id=((me + j) % N_DEV,),
                device_id_type=pl.DeviceIdType.MESH,
            )
        pl.semaphore_wait(barrier, N_DEV - 1)

        xb = x_ref[...].astype(jnp.bfloat16)
        wg = wg_ref[...].astype(jnp.bfloat16)
        wu = wu_ref[...].astype(jnp.bfloat16)
        wd = wd_ref[...].astype(jnp.bfloat16)

        for b in range(N_BLOCKS):
            xs = xb[BLOCK * b:BLOCK * (b + 1), :]
            gate = jnp.dot(xs, wg, preferred_element_type=jnp.float32)
            up = jnp.dot(xs, wu, preferred_element_type=jnp.float32)
            h = (gate * (up * jax.nn.sigmoid(up))).astype(jnp.bfloat16)
            pb = jnp.dot(h, wd, preferred_element_type=jnp.float32)
            stage[BLOCK * b:BLOCK * (b + 1), :] = pb.astype(jnp.bfloat16)
            for t in range(CPB * b, CPB * (b + 1)):
                slot = jnp.minimum((t - me - 1) % N_DEV, N_DEV - 2)

                @pl.when(t != me)
                def _():
                    rdma = pltpu.make_async_remote_copy(
                        src_ref=stage.at[pl.ds(t * CHUNK, CHUNK)],
                        dst_ref=rs_buf.at[slot],
                        send_sem=rs_send.at[slot],
                        recv_sem=rs_recv.at[slot],
                        device_id=(t,),
                        device_id_type=pl.DeviceIdType.MESH,
                    )
                    rdma.start()

        for k in range(N_DEV - 1):
            pltpu.make_async_remote_copy(
                src_ref=rs_buf.at[k],
                dst_ref=rs_buf.at[k],
                send_sem=rs_send.at[k],
                recv_sem=rs_recv.at[k],
                device_id=(me,),
                device_id_type=pl.DeviceIdType.MESH,
            ).wait_recv()

        my_lo = me * CHUNK
        reduced = stage[pl.ds(my_lo, CHUNK), :].astype(jnp.float32) + jnp.sum(
            rs_buf[...].astype(jnp.float32), axis=0
        )
        ag_buf[pl.ds(my_lo, CHUNK), :] = reduced.astype(jnp.bfloat16)
        for j in range(1, N_DEV):
            t = (me + j) % N_DEV
            rdma = pltpu.make_async_remote_copy(
                src_ref=ag_buf.at[pl.ds(my_lo, CHUNK)],
                dst_ref=ag_buf.at[pl.ds(my_lo, CHUNK)],
                send_sem=ag_send.at[j - 1],
                recv_sem=ag_recv.at[j - 1],
                device_id=(t,),
                device_id_type=pl.DeviceIdType.MESH,
            )
            rdma.start()

        for k in range(N_DEV - 1):
            s = (me + k + 1) % N_DEV
            pltpu.make_async_remote_copy(
                src_ref=ag_buf.at[pl.ds(s * CHUNK, CHUNK)],
                dst_ref=ag_buf.at[pl.ds(s * CHUNK, CHUNK)],
                send_sem=ag_send.at[k],
                recv_sem=ag_recv.at[k],
                device_id=(me,),
                device_id_type=pl.DeviceIdType.MESH,
            ).wait_recv()

        out_ref[...] = ag_buf[...].astype(jnp.float32)

        for k in range(N_DEV - 1):
            pltpu.make_async_remote_copy(
                src_ref=stage.at[pl.ds(0, CHUNK)],
                dst_ref=rs_buf.at[k],
                send_sem=rs_send.at[k],
                recv_sem=rs_recv.at[k],
                device_id=(me,),
                device_id_type=pl.DeviceIdType.MESH,
            ).wait_send()
            pltpu.make_async_remote_copy(
                src_ref=ag_buf.at[pl.ds(0, CHUNK)],
                dst_ref=ag_buf.at[pl.ds(0, CHUNK)],
                send_sem=ag_send.at[k],
                recv_sem=ag_recv.at[k],
                device_id=(me,),
                device_id_type=pl.DeviceIdType.MESH,
            ).wait_send()

    return pl.pallas_call(
        body,
        out_shape=jax.ShapeDtypeStruct((M, N_OUT), jnp.float32),
        in_specs=[pl.BlockSpec(memory_space=pltpu.VMEM)] * 4,
        out_specs=pl.BlockSpec(memory_space=pltpu.VMEM),
        scratch_shapes=[
            pltpu.VMEM((M, N_OUT), jnp.bfloat16),
            pltpu.VMEM((N_DEV - 1, CHUNK, N_OUT), jnp.bfloat16),
            pltpu.VMEM((M, N_OUT), jnp.bfloat16),
            pltpu.SemaphoreType.DMA((N_DEV - 1,)),
            pltpu.SemaphoreType.DMA((N_DEV - 1,)),
            pltpu.SemaphoreType.DMA((N_DEV - 1,)),
            pltpu.SemaphoreType.DMA((N_DEV - 1,)),
        ],
        compiler_params=pltpu.CompilerParams(collective_id=0),
    )(x, Wg, Wu, Wd)


# device time: 45403 ns/iter; 1.0999x vs baseline; 1.0999x over previous
import jax
import jax.numpy as jnp
from jax import lax
from jax.experimental import pallas as pl
from jax.experimental.pallas import tpu as pltpu

N_DEV = 16
M = 768
N_OUT = 768
CHUNK = M // N_DEV
N_BLOCKS = 4
BLOCK = M // N_BLOCKS
CPB = N_DEV // N_BLOCKS


def kernel(x, Wg, Wu, Wd):
    def body(x_ref, wg_ref, wu_ref, wd_ref, out_ref,
             stage, rs_buf, ag_buf,
             rs_send, rs_recv, ag_send, ag_recv):
        me = lax.axis_index("i")

        barrier = pltpu.get_barrier_semaphore()
        for j in range(1, N_DEV):
            pl.semaphore_signal(
                barrier, inc=1, device_id=((me + j) % N_DEV,),
                device_id_type=pl.DeviceIdType.MESH,
            )
        pl.semaphore_wait(barrier, N_DEV - 1)

        xb = x_ref[...].astype(jnp.bfloat16)
        gate = jnp.dot(xb, wg_ref[...].astype(jnp.bfloat16),
                       preferred_element_type=jnp.float32)
        up = jnp.dot(xb, wu_ref[...].astype(jnp.bfloat16),
                     preferred_element_type=jnp.float32)
        h = (gate * (up * jax.nn.sigmoid(up))).astype(jnp.bfloat16)
        wd = wd_ref[...].astype(jnp.bfloat16)

        for b in range(N_BLOCKS):
            pb = jnp.dot(h[BLOCK * b:BLOCK * (b + 1), :], wd,
                         preferred_element_type=jnp.float32)
            stage[BLOCK * b:BLOCK * (b + 1), :] = pb.astype(jnp.bfloat16)
            for t in range(CPB * b, CPB * (b + 1)):
                slot = jnp.minimum((t - me - 1) % N_DEV, N_DEV - 2)

                @pl.when(t != me)
                def _():
                    rdma = pltpu.make_async_remote_copy(
                        src_ref=stage.at[pl.ds(t * CHUNK, CHUNK)],
                        dst_ref=rs_buf.at[slot],
                        send_sem=rs_send.at[slot],
                        recv_sem=rs_recv.at[slot],
                        device_id=(t,),
                        device_id_type=pl.DeviceIdType.MESH,
                    )
                    rdma.start()

        for k in range(N_DEV - 1):
            pltpu.make_async_remote_copy(
                src_ref=rs_buf.at[k],
                dst_ref=rs_buf.at[k],
                send_sem=rs_send.at[k],
                recv_sem=rs_recv.at[k],
                device_id=(me,),
                device_id_type=pl.DeviceIdType.MESH,
            ).wait_recv()

        my_lo = me * CHUNK
        reduced = stage[pl.ds(my_lo, CHUNK), :].astype(jnp.float32) + jnp.sum(
            rs_buf[...].astype(jnp.float32), axis=0
        )
        ag_buf[pl.ds(my_lo, CHUNK), :] = reduced.astype(jnp.bfloat16)
        for j in range(1, N_DEV):
            t = (me + j) % N_DEV
            rdma = pltpu.make_async_remote_copy(
                src_ref=ag_buf.at[pl.ds(my_lo, CHUNK)],
                dst_ref=ag_buf.at[pl.ds(my_lo, CHUNK)],
                send_sem=ag_send.at[j - 1],
                recv_sem=ag_recv.at[j - 1],
                device_id=(t,),
                device_id_type=pl.DeviceIdType.MESH,
            )
            rdma.start()
        out_ref[pl.ds(my_lo, CHUNK), :] = reduced

        for k in range(N_DEV - 1):
            s = (me + k + 1) % N_DEV
            pltpu.make_async_remote_copy(
                src_ref=ag_buf.at[pl.ds(s * CHUNK, CHUNK)],
                dst_ref=ag_buf.at[pl.ds(s * CHUNK, CHUNK)],
                send_sem=ag_send.at[k],
                recv_sem=ag_recv.at[k],
                device_id=(me,),
                device_id_type=pl.DeviceIdType.MESH,
            ).wait_recv()
            out_ref[pl.ds(s * CHUNK, CHUNK), :] = (
                ag_buf[pl.ds(s * CHUNK, CHUNK), :].astype(jnp.float32)
            )

        for k in range(N_DEV - 1):
            pltpu.make_async_remote_copy(
                src_ref=stage.at[pl.ds(0, CHUNK)],
                dst_ref=rs_buf.at[k],
                send_sem=rs_send.at[k],
                recv_sem=rs_recv.at[k],
                device_id=(me,),
                device_id_type=pl.DeviceIdType.MESH,
            ).wait_send()
            pltpu.make_async_remote_copy(
                src_ref=ag_buf.at[pl.ds(0, CHUNK)],
                dst_ref=ag_buf.at[pl.ds(0, CHUNK)],
                send_sem=ag_send.at[k],
                recv_sem=ag_recv.at[k],
                device_id=(me,),
                device_id_type=pl.DeviceIdType.MESH,
            ).wait_send()

    return pl.pallas_call(
        body,
        out_shape=jax.ShapeDtypeStruct((M, N_OUT), jnp.float32),
        in_specs=[pl.BlockSpec(memory_space=pltpu.VMEM)] * 4,
        out_specs=pl.BlockSpec(memory_space=pltpu.VMEM),
        scratch_shapes=[
            pltpu.VMEM((M, N_OUT), jnp.bfloat16),
            pltpu.VMEM((N_DEV - 1, CHUNK, N_OUT), jnp.bfloat16),
            pltpu.VMEM((M, N_OUT), jnp.bfloat16),
            pltpu.SemaphoreType.DMA((N_DEV - 1,)),
            pltpu.SemaphoreType.DMA((N_DEV - 1,)),
            pltpu.SemaphoreType.DMA((N_DEV - 1,)),
            pltpu.SemaphoreType.DMA((N_DEV - 1,)),
        ],
        compiler_params=pltpu.CompilerParams(collective_id=0),
    )(x, Wg, Wu, Wd)
